# Initial kernel scaffold; baseline (speedup 1.0000x reference)
import jax
import jax.numpy as jnp
from jax.experimental import pallas as pl


def kernel(actions, att_mask, action_table, pos_table, ln_weight, ln_bias):
    raise NotImplementedError("write your pallas kernel here")


# speedup vs baseline: 1.4096x; 1.4096x over previous
import jax
import jax.numpy as jnp
import numpy as np
from jax.experimental import pallas as pl

D_MODEL = 128
NUM_ACTIONS = 1000
VOCAB = NUM_ACTIONS + 1
SEQ = 201
EPS = 1e-12
SQRT_D = np.sqrt(D_MODEL)

B_TILE = 16


def _embed_ln_kernel(idx_ref, hi_ref, lo_ref, pos_ref, w_ref, b_ref, out_ref):
    n_tok = B_TILE * SEQ
    idx = idx_ref[...]
    iota = jax.lax.broadcasted_iota(jnp.int16, (n_tok, VOCAB), 1)
    onehot = jnp.where(
        idx.astype(jnp.int16) == iota, jnp.bfloat16(1), jnp.bfloat16(0)
    )
    g = jnp.dot(onehot, hi_ref[...], preferred_element_type=jnp.float32)
    g += jnp.dot(onehot, lo_ref[...], preferred_element_type=jnp.float32)
    x = g.reshape(B_TILE, SEQ, D_MODEL) * SQRT_D
    x = x + pos_ref[...][None, :, :]
    mean = jnp.mean(x, axis=-1, keepdims=True)
    var = jnp.mean(jnp.square(x - mean), axis=-1, keepdims=True)
    normed = (x - mean) * jax.lax.rsqrt(var + EPS)
    out_ref[...] = normed * w_ref[...][None] + b_ref[...][None]


@jax.jit
def kernel(actions, att_mask, action_table, pos_table, ln_weight, ln_bias):
    batch = actions.shape[0]
    cls_col = jnp.full((batch, 1), NUM_ACTIONS, dtype=actions.dtype)
    acts = jnp.concatenate([cls_col, actions], axis=1)
    mask = jnp.concatenate(
        [jnp.zeros((batch, 1), dtype=att_mask.dtype), att_mask], axis=1
    )

    t_hi = action_table.astype(jnp.bfloat16)
    t_lo = (action_table - t_hi.astype(jnp.float32)).astype(jnp.bfloat16)

    n_tok = batch * SEQ
    idx_col = acts.reshape(n_tok, 1)
    grid = (batch // B_TILE,)
    tile_tok = B_TILE * SEQ

    out = pl.pallas_call(
        _embed_ln_kernel,
        grid=grid,
        in_specs=[
            pl.BlockSpec((tile_tok, 1), lambda i: (i, 0)),
            pl.BlockSpec((VOCAB, D_MODEL), lambda i: (0, 0)),
            pl.BlockSpec((VOCAB, D_MODEL), lambda i: (0, 0)),
            pl.BlockSpec((SEQ, D_MODEL), lambda i: (0, 0)),
            pl.BlockSpec((1, D_MODEL), lambda i: (0, 0)),
            pl.BlockSpec((1, D_MODEL), lambda i: (0, 0)),
        ],
        out_specs=pl.BlockSpec((B_TILE, SEQ, D_MODEL), lambda i: (i, 0, 0)),
        out_shape=jax.ShapeDtypeStruct((batch, SEQ, D_MODEL), jnp.float32),
    )(
        idx_col,
        t_hi,
        t_lo,
        pos_table,
        ln_weight.reshape(1, D_MODEL),
        ln_bias.reshape(1, D_MODEL),
    )
    return (out, mask)


# speedup vs baseline: 3.1819x; 2.2573x over previous
import functools

import jax
import jax.numpy as jnp
import numpy as np
from jax import lax
from jax.experimental import pallas as pl
from jax.experimental.pallas import tpu as pltpu
from jax.experimental.pallas import tpu_sc as plsc

D_MODEL = 128
NUM_ACTIONS = 1000
VOCAB = NUM_ACTIONS + 1
SEQ = 201
EPS = 1e-12
SQRT_D = np.sqrt(D_MODEL)

A_TILE = 16
PID_TILE = 256

BATCH = 4096
N_TOK = BATCH * SEQ
NUM_WORKERS = 32
PER_W = N_TOK // NUM_WORKERS
CHUNK = 536
STEPS = PER_W // CHUNK


def _pairtab_kernel(act_ref, pos_ref, w_ref, b_ref, out_ref):
    x = act_ref[...][:, None, :] * SQRT_D + pos_ref[...][None, :, :]
    mean = jnp.mean(x, axis=-1, keepdims=True)
    var = jnp.mean(jnp.square(x - mean), axis=-1, keepdims=True)
    normed = (x - mean) * jax.lax.rsqrt(var + EPS)
    out_ref[...] = normed * w_ref[...][None] + b_ref[...][None]


def _pid_kernel(acts_ref, out_ref):
    s = jax.lax.broadcasted_iota(jnp.int32, (PID_TILE, SEQ), 1)
    out_ref[...] = acts_ref[...] * SEQ + s


def _sc_gather_kernel(tab_hbm, pid_hbm, out_hbm, idx_v, rows_v, sem):
    wid = lax.axis_index("s") * 2 + lax.axis_index("c")
    base = wid * PER_W

    @pl.loop(0, STEPS)
    def _(i):
        off = base + i * CHUNK
        pltpu.sync_copy(pid_hbm.at[pl.ds(off, CHUNK)], idx_v)
        pltpu.async_copy(tab_hbm.at[idx_v], rows_v, sem).wait()
        pltpu.sync_copy(rows_v, out_hbm.at[pl.ds(off, CHUNK)])


@jax.jit
def kernel(actions, att_mask, action_table, pos_table, ln_weight, ln_bias):
    batch = actions.shape[0]
    cls_col = jnp.full((batch, 1), NUM_ACTIONS, dtype=actions.dtype)
    acts = jnp.concatenate([cls_col, actions], axis=1)
    mask = jnp.concatenate(
        [jnp.zeros((batch, 1), dtype=att_mask.dtype), att_mask], axis=1
    )

    pairtab = pl.pallas_call(
        _pairtab_kernel,
        grid=(pl.cdiv(VOCAB, A_TILE),),
        in_specs=[
            pl.BlockSpec((A_TILE, D_MODEL), lambda i: (i, 0)),
            pl.BlockSpec((SEQ, D_MODEL), lambda i: (0, 0)),
            pl.BlockSpec((1, D_MODEL), lambda i: (0, 0)),
            pl.BlockSpec((1, D_MODEL), lambda i: (0, 0)),
        ],
        out_specs=pl.BlockSpec((A_TILE, SEQ, D_MODEL), lambda i: (i, 0, 0)),
        out_shape=jax.ShapeDtypeStruct((VOCAB, SEQ, D_MODEL), jnp.float32),
    )(
        action_table,
        pos_table,
        ln_weight.reshape(1, D_MODEL),
        ln_bias.reshape(1, D_MODEL),
    )
    tab_flat = pairtab.reshape(VOCAB * SEQ, D_MODEL)

    pid = pl.pallas_call(
        _pid_kernel,
        grid=(batch // PID_TILE,),
        in_specs=[pl.BlockSpec((PID_TILE, SEQ), lambda i: (i, 0))],
        out_specs=pl.BlockSpec((PID_TILE, SEQ), lambda i: (i, 0)),
        out_shape=jax.ShapeDtypeStruct((batch, SEQ), jnp.int32),
    )(acts)
    pid_flat = pid.reshape(N_TOK)

    mesh = plsc.VectorSubcoreMesh(core_axis_name="c", subcore_axis_name="s")
    sc_gather = functools.partial(
        pl.kernel,
        mesh=mesh,
        out_type=jax.ShapeDtypeStruct((N_TOK, D_MODEL), jnp.float32),
        scratch_types=[
            pltpu.VMEM((CHUNK,), jnp.int32),
            pltpu.VMEM((CHUNK, D_MODEL), jnp.float32),
            pltpu.SemaphoreType.DMA,
        ],
    )(_sc_gather_kernel)
    out_flat = sc_gather(tab_flat, pid_flat)

    out = out_flat.reshape(batch, SEQ, D_MODEL)
    return (out, mask)


# speedup vs baseline: 4.1292x; 1.2977x over previous
import functools

import jax
import jax.numpy as jnp
import numpy as np
from jax import lax
from jax.experimental import pallas as pl
from jax.experimental.pallas import tpu as pltpu
from jax.experimental.pallas import tpu_sc as plsc

D_MODEL = 128
NUM_ACTIONS = 1000
VOCAB = NUM_ACTIONS + 1
SEQ = 201
SEQ_PAD = 256
EPS = 1e-12
SQRT_D = np.sqrt(D_MODEL)

A_TILE = 16
PID_TILE = 256

BATCH = 4096
NUM_WORKERS = 32
ROWS_PW = BATCH // NUM_WORKERS


def _pairtab_kernel(act_ref, pos_ref, w_ref, b_ref, out_ref):
    x = act_ref[...][:, None, :] * SQRT_D + pos_ref[...][None, :, :]
    mean = jnp.mean(x, axis=-1, keepdims=True)
    var = jnp.mean(jnp.square(x - mean), axis=-1, keepdims=True)
    normed = (x - mean) * jax.lax.rsqrt(var + EPS)
    res = normed * w_ref[...][None] + b_ref[...][None]
    out_ref[...] = res.reshape(A_TILE * SEQ, D_MODEL)


def _pid_kernel(acts_ref, out_ref):
    s = jax.lax.broadcasted_iota(jnp.int32, (PID_TILE, SEQ_PAD), 1)
    pid = acts_ref[...] * SEQ + s
    out_ref[...] = jnp.where(s < SEQ, pid, s)


def _sc_gather_kernel(
    tab_hbm,
    pid_hbm,
    out_hbm,
    idx0,
    idx1,
    rows0,
    rows1,
    si0,
    si1,
    sg0,
    sg1,
):
    wid = lax.axis_index("s") * 2 + lax.axis_index("c")
    row0 = wid * ROWS_PW

    pltpu.async_copy(pid_hbm.at[row0], idx0, si0)
    pltpu.async_copy(pid_hbm.at[row0 + 1], idx1, si1)
    pltpu.make_async_copy(pid_hbm.at[row0], idx0, si0).wait()
    pltpu.async_copy(tab_hbm.at[idx0], rows0, sg0)

    @pl.loop(0, ROWS_PW, step=2)
    def _(r):
        pltpu.make_async_copy(pid_hbm.at[row0 + r + 1], idx1, si1).wait()
        pltpu.make_async_copy(tab_hbm.at[idx0], rows0, sg0).wait()
        pltpu.async_copy(tab_hbm.at[idx1], rows1, sg1)

        @pl.when(r + 2 < ROWS_PW)
        def _():
            pltpu.async_copy(pid_hbm.at[row0 + r + 2], idx0, si0)

        pltpu.sync_copy(rows0.at[pl.ds(0, SEQ)], out_hbm.at[row0 + r])

        @pl.when(r + 2 < ROWS_PW)
        def _():
            pltpu.make_async_copy(pid_hbm.at[row0 + r + 2], idx0, si0).wait()

        pltpu.make_async_copy(tab_hbm.at[idx1], rows1, sg1).wait()

        @pl.when(r + 2 < ROWS_PW)
        def _():
            pltpu.async_copy(tab_hbm.at[idx0], rows0, sg0)

        @pl.when(r + 3 < ROWS_PW)
        def _():
            pltpu.async_copy(pid_hbm.at[row0 + r + 3], idx1, si1)

        pltpu.sync_copy(rows1.at[pl.ds(0, SEQ)], out_hbm.at[row0 + r + 1])


@jax.jit
def kernel(actions, att_mask, action_table, pos_table, ln_weight, ln_bias):
    batch = actions.shape[0]
    cls_col = jnp.full((batch, 1), NUM_ACTIONS, dtype=actions.dtype)
    acts = jnp.concatenate([cls_col, actions], axis=1)
    acts_pad = jnp.pad(acts, ((0, 0), (0, SEQ_PAD - SEQ)))
    mask = jnp.concatenate(
        [jnp.zeros((batch, 1), dtype=att_mask.dtype), att_mask], axis=1
    )

    tab_flat = pl.pallas_call(
        _pairtab_kernel,
        grid=(pl.cdiv(VOCAB, A_TILE),),
        in_specs=[
            pl.BlockSpec((A_TILE, D_MODEL), lambda i: (i, 0)),
            pl.BlockSpec((SEQ, D_MODEL), lambda i: (0, 0)),
            pl.BlockSpec((1, D_MODEL), lambda i: (0, 0)),
            pl.BlockSpec((1, D_MODEL), lambda i: (0, 0)),
        ],
        out_specs=pl.BlockSpec((A_TILE * SEQ, D_MODEL), lambda i: (i, 0)),
        out_shape=jax.ShapeDtypeStruct((VOCAB * SEQ, D_MODEL), jnp.float32),
    )(
        action_table,
        pos_table,
        ln_weight.reshape(1, D_MODEL),
        ln_bias.reshape(1, D_MODEL),
    )

    pid = pl.pallas_call(
        _pid_kernel,
        grid=(batch // PID_TILE,),
        in_specs=[pl.BlockSpec((PID_TILE, SEQ_PAD), lambda i: (i, 0))],
        out_specs=pl.BlockSpec((PID_TILE, SEQ_PAD), lambda i: (i, 0)),
        out_shape=jax.ShapeDtypeStruct((batch, SEQ_PAD), jnp.int32),
    )(acts_pad)

    mesh = plsc.VectorSubcoreMesh(core_axis_name="c", subcore_axis_name="s")
    sc_gather = functools.partial(
        pl.kernel,
        mesh=mesh,
        out_type=jax.ShapeDtypeStruct((batch, SEQ, D_MODEL), jnp.float32),
        scratch_types=[
            pltpu.VMEM((SEQ_PAD,), jnp.int32),
            pltpu.VMEM((SEQ_PAD,), jnp.int32),
            pltpu.VMEM((SEQ_PAD, D_MODEL), jnp.float32),
            pltpu.VMEM((SEQ_PAD, D_MODEL), jnp.float32),
            pltpu.SemaphoreType.DMA,
            pltpu.SemaphoreType.DMA,
            pltpu.SemaphoreType.DMA,
            pltpu.SemaphoreType.DMA,
        ],
    )(_sc_gather_kernel)
    out = sc_gather(tab_flat, pid)

    return (out, mask)


# speedup vs baseline: 4.5330x; 1.0978x over previous
import functools

import jax
import jax.numpy as jnp
import numpy as np
from jax import lax
from jax.experimental import pallas as pl
from jax.experimental.pallas import tpu as pltpu
from jax.experimental.pallas import tpu_sc as plsc

D_MODEL = 128
NUM_ACTIONS = 1000
VOCAB = NUM_ACTIONS + 1
SEQ = 201
SEQ_PAD = 208
EPS = 1e-12
SQRT_D = np.sqrt(D_MODEL)

A_TILE = 16
PID_TILE = 256

BATCH = 4096
NUM_WORKERS = 32
ROWS_PW = BATCH // NUM_WORKERS


def _pairtab_kernel(act_ref, pos_ref, w_ref, b_ref, out_ref):
    x = act_ref[...][:, None, :] * SQRT_D + pos_ref[...][None, :, :]
    mean = jnp.mean(x, axis=-1, keepdims=True)
    var = jnp.mean(jnp.square(x - mean), axis=-1, keepdims=True)
    normed = (x - mean) * jax.lax.rsqrt(var + EPS)
    res = normed * w_ref[...][None] + b_ref[...][None]
    out_ref[...] = res.reshape(A_TILE * SEQ_PAD, D_MODEL)


def _pid_kernel(acts_ref, out_ref):
    s = jax.lax.broadcasted_iota(jnp.int32, (PID_TILE, SEQ_PAD), 1)
    out_ref[...] = jnp.where(s < SEQ, acts_ref[...] * SEQ_PAD + s, s)


def _sc_gather_kernel(tab_hbm, pid_hbm, out_hbm, idx, rows, si, sg, sw):
    wid = lax.axis_index("s") * 2 + lax.axis_index("c")
    row0 = wid * ROWS_PW

    for k in range(4):
        pltpu.async_copy(pid_hbm.at[row0 + k], idx[k], si[k])
    for k in range(2):
        pltpu.make_async_copy(pid_hbm.at[row0 + k], idx[k], si[k]).wait()
        pltpu.async_copy(tab_hbm.at[idx[k]], rows[k], sg[k])

    @pl.loop(0, ROWS_PW, step=4)
    def _(r):
        for k in range(4):
            rr = r + k
            k2 = (k + 2) % 4
            @pl.when(rr + 2 < ROWS_PW)
            def _():
                pltpu.make_async_copy(
                    pid_hbm.at[row0 + rr + 2], idx[k2], si[k2]
                ).wait()

            @pl.when(jnp.logical_and(rr + 2 < ROWS_PW, rr >= 2))
            def _():
                pltpu.make_async_copy(
                    rows[k2].at[pl.ds(0, SEQ)],
                    out_hbm.at[row0 + rr - 2],
                    sw[k2],
                ).wait()

            @pl.when(rr + 2 < ROWS_PW)
            def _():
                pltpu.async_copy(tab_hbm.at[idx[k2]], rows[k2], sg[k2])

            pltpu.make_async_copy(tab_hbm.at[idx[k]], rows[k], sg[k]).wait()

            @pl.when(rr + 4 < ROWS_PW)
            def _():
                pltpu.async_copy(pid_hbm.at[row0 + rr + 4], idx[k], si[k])

            pltpu.async_copy(
                rows[k].at[pl.ds(0, SEQ)], out_hbm.at[row0 + rr], sw[k]
            )

    for k in range(4):
        rr_last = ROWS_PW - 4 + k
        pltpu.make_async_copy(
            rows[k].at[pl.ds(0, SEQ)], out_hbm.at[row0 + rr_last], sw[k]
        ).wait()


def _sc_gather_body(
    tab_hbm,
    pid_hbm,
    out_hbm,
    i0,
    i1,
    i2,
    i3,
    r0,
    r1,
    r2,
    r3,
    si0,
    si1,
    si2,
    si3,
    sg0,
    sg1,
    sg2,
    sg3,
    sw0,
    sw1,
    sw2,
    sw3,
):
    _sc_gather_kernel(
        tab_hbm,
        pid_hbm,
        out_hbm,
        [i0, i1, i2, i3],
        [r0, r1, r2, r3],
        [si0, si1, si2, si3],
        [sg0, sg1, sg2, sg3],
        [sw0, sw1, sw2, sw3],
    )


@jax.jit
def kernel(actions, att_mask, action_table, pos_table, ln_weight, ln_bias):
    batch = actions.shape[0]
    cls_col = jnp.full((batch, 1), NUM_ACTIONS, dtype=actions.dtype)
    acts = jnp.concatenate([cls_col, actions], axis=1)
    acts_pad = jnp.pad(acts, ((0, 0), (0, SEQ_PAD - SEQ)))
    pos_pad = jnp.pad(pos_table, ((0, SEQ_PAD - SEQ), (0, 0)))
    mask = jnp.concatenate(
        [jnp.zeros((batch, 1), dtype=att_mask.dtype), att_mask], axis=1
    )

    tab_flat = pl.pallas_call(
        _pairtab_kernel,
        grid=(pl.cdiv(VOCAB, A_TILE),),
        in_specs=[
            pl.BlockSpec((A_TILE, D_MODEL), lambda i: (i, 0)),
            pl.BlockSpec((SEQ_PAD, D_MODEL), lambda i: (0, 0)),
            pl.BlockSpec((1, D_MODEL), lambda i: (0, 0)),
            pl.BlockSpec((1, D_MODEL), lambda i: (0, 0)),
        ],
        out_specs=pl.BlockSpec((A_TILE * SEQ_PAD, D_MODEL), lambda i: (i, 0)),
        out_shape=jax.ShapeDtypeStruct((VOCAB * SEQ_PAD, D_MODEL), jnp.float32),
    )(
        action_table,
        pos_pad,
        ln_weight.reshape(1, D_MODEL),
        ln_bias.reshape(1, D_MODEL),
    )

    pid = pl.pallas_call(
        _pid_kernel,
        grid=(batch // PID_TILE,),
        in_specs=[pl.BlockSpec((PID_TILE, SEQ_PAD), lambda i: (i, 0))],
        out_specs=pl.BlockSpec((PID_TILE, SEQ_PAD), lambda i: (i, 0)),
        out_shape=jax.ShapeDtypeStruct((batch, SEQ_PAD), jnp.int32),
    )(acts_pad)

    mesh = plsc.VectorSubcoreMesh(core_axis_name="c", subcore_axis_name="s")
    sc_gather = functools.partial(
        pl.kernel,
        mesh=mesh,
        out_type=jax.ShapeDtypeStruct((batch, SEQ, D_MODEL), jnp.float32),
        scratch_types=(
            [pltpu.VMEM((SEQ_PAD,), jnp.int32)] * 4
            + [pltpu.VMEM((SEQ_PAD, D_MODEL), jnp.float32)] * 4
            + [pltpu.SemaphoreType.DMA] * 12
        ),
    )(_sc_gather_body)
    out = sc_gather(tab_flat, pid)

    return (out, mask)


# speedup vs baseline: 4.5794x; 1.0102x over previous
import functools

import jax
import jax.numpy as jnp
import numpy as np
from jax import lax
from jax.experimental import pallas as pl
from jax.experimental.pallas import tpu as pltpu
from jax.experimental.pallas import tpu_sc as plsc

D_MODEL = 128
NUM_ACTIONS = 1000
VOCAB = NUM_ACTIONS + 1
SEQ = 201
SEQ_PAD = 208
EPS = 1e-12
SQRT_D = np.sqrt(D_MODEL)

A_TILE = 16
PID_TILE = 256

BATCH = 4096
NUM_WORKERS = 32
ROWS_PW = BATCH // NUM_WORKERS


def _pairtab_kernel(act_ref, pos_ref, w_ref, b_ref, out_ref):
    x = act_ref[...][:, None, :] * SQRT_D + pos_ref[...][None, :, :]
    mean = jnp.mean(x, axis=-1, keepdims=True)
    var = jnp.mean(jnp.square(x - mean), axis=-1, keepdims=True)
    normed = (x - mean) * jax.lax.rsqrt(var + EPS)
    res = normed * w_ref[...][None] + b_ref[...][None]
    out_ref[...] = res.reshape(A_TILE * SEQ_PAD, D_MODEL)


def _pid_kernel(acts_ref, out_ref):
    c = jax.lax.broadcasted_iota(jnp.int32, (PID_TILE, 2 * SEQ), 1)
    s = jnp.where(c < SEQ, c, c - SEQ)
    out_ref[...] = acts_ref[...] * SEQ_PAD + s


NIDX = 8
PAIRS_PW = ROWS_PW // 2
CHUNK = 2 * SEQ


def _sc_gather_kernel(tab_hbm, pid_hbm, out_hbm, idx, rows, si, sg, sw):
    wid = lax.axis_index("s") * 2 + lax.axis_index("c")
    pr0 = wid * PAIRS_PW

    for k in range(4):
        pltpu.async_copy(pid_hbm.at[pr0 + k], idx[k], si[k])

    @pl.loop(0, PAIRS_PW, step=NIDX)
    def _(r):
        for k in range(NIDX):
            rr = r + k
            j = k % 2
            b0 = (pr0 + rr) * 2

            @pl.when(rr >= 2)
            def _():
                pltpu.make_async_copy(
                    rows[j].at[pl.ds(0, SEQ)], out_hbm.at[b0 - 4], sw[j]
                ).wait()
                pltpu.make_async_copy(
                    rows[j].at[pl.ds(SEQ, SEQ)], out_hbm.at[b0 - 3], sw[j]
                ).wait()

            pltpu.make_async_copy(pid_hbm.at[pr0 + rr], idx[k], si[k]).wait()
            pltpu.async_copy(tab_hbm.at[idx[k]], rows[j], sg[j])
            pltpu.make_async_copy(tab_hbm.at[idx[k]], rows[j], sg[j]).wait()

            kf = (k + 4) % NIDX

            @pl.when(rr + 4 < PAIRS_PW)
            def _():
                pltpu.async_copy(pid_hbm.at[pr0 + rr + 4], idx[kf], si[kf])

            pltpu.async_copy(
                rows[j].at[pl.ds(0, SEQ)], out_hbm.at[b0], sw[j]
            )
            pltpu.async_copy(
                rows[j].at[pl.ds(SEQ, SEQ)], out_hbm.at[b0 + 1], sw[j]
            )

    for j in range(2):
        rr = PAIRS_PW - 2 + j
        b0 = (pr0 + rr) * 2
        pltpu.make_async_copy(
            rows[j].at[pl.ds(0, SEQ)], out_hbm.at[b0], sw[j]
        ).wait()
        pltpu.make_async_copy(
            rows[j].at[pl.ds(SEQ, SEQ)], out_hbm.at[b0 + 1], sw[j]
        ).wait()


def _sc_gather_body(tab_hbm, pid_hbm, out_hbm, *rest):
    idx = list(rest[:NIDX])
    rows = list(rest[NIDX : NIDX + 2])
    si = list(rest[NIDX + 2 : 2 * NIDX + 2])
    sg = list(rest[2 * NIDX + 2 : 2 * NIDX + 4])
    sw = list(rest[2 * NIDX + 4 : 2 * NIDX + 6])
    _sc_gather_kernel(tab_hbm, pid_hbm, out_hbm, idx, rows, si, sg, sw)


@jax.jit
def kernel(actions, att_mask, action_table, pos_table, ln_weight, ln_bias):
    batch = actions.shape[0]
    cls_col = jnp.full((batch, 1), NUM_ACTIONS, dtype=actions.dtype)
    acts = jnp.concatenate([cls_col, actions], axis=1)
    pos_pad = jnp.pad(pos_table, ((0, SEQ_PAD - SEQ), (0, 0)))
    mask = jnp.concatenate(
        [jnp.zeros((batch, 1), dtype=att_mask.dtype), att_mask], axis=1
    )

    tab_flat = pl.pallas_call(
        _pairtab_kernel,
        grid=(pl.cdiv(VOCAB, A_TILE),),
        in_specs=[
            pl.BlockSpec((A_TILE, D_MODEL), lambda i: (i, 0)),
            pl.BlockSpec((SEQ_PAD, D_MODEL), lambda i: (0, 0)),
            pl.BlockSpec((1, D_MODEL), lambda i: (0, 0)),
            pl.BlockSpec((1, D_MODEL), lambda i: (0, 0)),
        ],
        out_specs=pl.BlockSpec((A_TILE * SEQ_PAD, D_MODEL), lambda i: (i, 0)),
        out_shape=jax.ShapeDtypeStruct((VOCAB * SEQ_PAD, D_MODEL), jnp.float32),
    )(
        action_table,
        pos_pad,
        ln_weight.reshape(1, D_MODEL),
        ln_bias.reshape(1, D_MODEL),
    )

    acts2 = acts.reshape(batch // 2, 2 * SEQ)
    pid = pl.pallas_call(
        _pid_kernel,
        grid=(batch // 2 // PID_TILE,),
        in_specs=[pl.BlockSpec((PID_TILE, 2 * SEQ), lambda i: (i, 0))],
        out_specs=pl.BlockSpec((PID_TILE, 2 * SEQ), lambda i: (i, 0)),
        out_shape=jax.ShapeDtypeStruct((batch // 2, 2 * SEQ), jnp.int32),
    )(acts2)

    mesh = plsc.VectorSubcoreMesh(core_axis_name="c", subcore_axis_name="s")
    sc_gather = functools.partial(
        pl.kernel,
        mesh=mesh,
        out_type=jax.ShapeDtypeStruct((batch, SEQ, D_MODEL), jnp.float32),
        scratch_types=(
            [pltpu.VMEM((CHUNK,), jnp.int32)] * NIDX
            + [pltpu.VMEM((CHUNK, D_MODEL), jnp.float32)] * 2
            + [pltpu.SemaphoreType.DMA] * (NIDX + 4)
        ),
    )(_sc_gather_body)
    out = sc_gather(tab_flat, pid)

    return (out, mask)
